# baseline (speedup 1.0000x reference)
import functools

import jax
import jax.numpy as jnp
from jax import lax
from jax.experimental import pallas as pl
from jax.experimental.pallas import tpu as pltpu
from jax.experimental.pallas import tpu_sc as plsc

N_TOTAL = 10000
M = 2
S = N_TOTAL * M
K = 8
SK = S * K
E = 320000
H = 128
IN_CH = 128
EDGE_DIM = 16
B_GRAPHS = 64

NC = 2
NS = 16
NW = NC * NS

R_ROWS = 10240
N_PASSES = -(-SK // (NC * R_ROWS))
TRASH = R_ROWS
E_TILE = E // NS
EC = 2000
NCHUNK = E_TILE // EC
QCAP = EC + 160
WBLK = 128

_mesh = plsc.VectorSubcoreMesh(core_axis_name="c", subcore_axis_name="s",
                               num_cores=NC, num_subcores=NS)


PRO_CHUNK = 200
PRO_PER_W = SK // NW
PRO_NCHUNK = PRO_PER_W // PRO_CHUNK


def _pro_body(comb_hbm, x_hbm, nid_hbm, h0_hbm, x_res, nid_st,
              idx0, idx1, rows0, rows1,
              semn, semg0, semg1, semw0, semw1):
    c = lax.axis_index("c")
    s = lax.axis_index("s")
    w = s * NC + c
    base = w * PRO_PER_W
    pltpu.sync_copy(x_hbm, x_res)
    iota16 = lax.iota(jnp.int32, 16)
    rootpat = (iota16 % K) == 0
    bufs = ((idx0, rows0, semg0, semw0), (idx1, rows1, semg1, semw1))

    def _nid_refs(ci):
        return (nid_hbm.at[pl.ds(base + ci * PRO_CHUNK, PRO_CHUNK)],
                nid_st.at[pl.ds((ci % 2) * 208, PRO_CHUNK)], semn)

    def _gather_cps(ci):
        idxb, rowsb, semg, _ = bufs[ci % 2]
        return ((comb_hbm.at[idxb.at[pl.ds(0, 128)]],
                 rowsb.at[pl.ds(0, 128)], semg),
                (comb_hbm.at[idxb.at[pl.ds(128, PRO_CHUNK - 128)]],
                 rowsb.at[pl.ds(128, PRO_CHUNK - 128)], semg))

    def _wr_refs(ci):
        rowsb, semw = bufs[ci % 2][1], bufs[ci % 2][3]
        return (rowsb, h0_hbm.at[pl.ds(base + ci * PRO_CHUNK, PRO_CHUNK)], semw)

    sre, dst0, sm = _nid_refs(0)
    pltpu.async_copy(sre, dst0, sm)
    for ci in range(PRO_NCHUNK):
        b = ci % 2
        idxb = bufs[b][0]
        sre, dstn, sm = _nid_refs(ci)
        pltpu.make_async_copy(sre, dstn, sm).wait()
        if ci + 1 < PRO_NCHUNK:
            sre, dstn, sm = _nid_refs(ci + 1)
            pltpu.async_copy(sre, dstn, sm)
        rep = (w % 8) * (2 * IN_CH)
        for g in range(13):
            nv = nid_st[pl.ds(b * 208 + g * 16, 16)]
            if g == 12:
                nv = jnp.where(iota16 < 8, nv, 0)
            xv = plsc.load_gather(x_res, [nv])
            idxb[pl.ds(g * 16, 16)] = xv + jnp.where(rootpat, IN_CH, 0) + rep
        if ci >= 2:
            sre, dstn, sm = _wr_refs(ci - 2)
            pltpu.make_async_copy(sre, dstn, sm).wait()
        for sre, dstn, sm in _gather_cps(ci):
            pltpu.async_copy(sre, dstn, sm)
        if ci >= 1:
            for sre, dstn, sm in _gather_cps(ci - 1):
                pltpu.make_async_copy(sre, dstn, sm).wait()
            sre, dstn, sm = _wr_refs(ci - 1)
            pltpu.async_copy(sre, dstn, sm)
    last = PRO_NCHUNK - 1
    for sre, dstn, sm in _gather_cps(last):
        pltpu.make_async_copy(sre, dstn, sm).wait()
    sre, dstn, sm = _wr_refs(last)
    pltpu.async_copy(sre, dstn, sm)
    for ci in (last - 1, last):
        sre, dstn, sm = _wr_refs(ci)
        pltpu.make_async_copy(sre, dstn, sm).wait()


_prologue = functools.partial(
    pl.kernel, _pro_body, mesh=_mesh,
    compiler_params=pltpu.CompilerParams(needs_layout_passes=False),
    out_type=jax.ShapeDtypeStruct((SK, H), jnp.float32),
    scratch_types=[
        pltpu.VMEM((N_TOTAL,), jnp.int32),
        pltpu.VMEM((416,), jnp.int32),
        pltpu.VMEM((PRO_CHUNK + 8,), jnp.int32),
        pltpu.VMEM((PRO_CHUNK + 8,), jnp.int32),
        pltpu.VMEM((PRO_CHUNK, H), jnp.float32),
        pltpu.VMEM((PRO_CHUNK, H), jnp.float32),
        pltpu.SemaphoreType.DMA,
        pltpu.SemaphoreType.DMA,
        pltpu.SemaphoreType.DMA,
        pltpu.SemaphoreType.DMA,
        pltpu.SemaphoreType.DMA,
    ])()


def _agg_body(h_hbm, se_hbm, dst_hbm, bond_hbm, agg_hbm,
              st_se, st_dst, q_se, q_loc, rows0, rows1, bond_v,
              sx0, sx1, lx0, lx1, ef0, ef1, acc_sh,
              semg0, semg1, semc0, semc1, sem_s1, sem_s2, sem_z, sem_w):
    c = lax.axis_index("c")
    s = lax.axis_index("s")
    ebase = s * E_TILE
    iota16 = lax.iota(jnp.int32, 16)
    pltpu.sync_copy(bond_hbm, bond_v)
    bufs = ((rows0, sx0, lx0, ef0, semg0, semc0),
            (rows1, sx1, lx1, ef1, semg1, semc1))

    def _stage_refs(ci):
        boff = (ci % 2) * EC
        sl = pl.ds(ebase + ci * EC, EC)
        bl = pl.ds(boff, EC)
        return ((se_hbm.at[sl], st_se.at[bl], sem_s1),
                (dst_hbm.at[sl], st_dst.at[bl], sem_s2))

    def _stage_issue(ci):
        for src, dst, sem in _stage_refs(ci):
            pltpu.async_copy(src, dst, sem)

    def _stage_wait(ci):
        for src, dst, sem in _stage_refs(ci):
            pltpu.make_async_copy(src, dst, sem).wait()

    def _wait_scatter(bi):
        rows_b, _, lx, _, _, semc = bufs[bi]
        pltpu.make_async_copy(rows_b, acc_sh.at[lx], semc).wait()

    def _issue(qbase, nvalid, j, bi):
        rows_b, sx, lx, ef, semg, _ = bufs[bi]

        @pl.when(j >= 2)
        def _():
            _wait_scatter(bi)
        for g in range(8):
            lane = g * 16 + iota16
            valid = lane < nvalid
            sv = q_se[pl.ds(qbase + g * 16, 16)]
            lg = q_loc[pl.ds(qbase + g * 16, 16)]
            sx[pl.ds(g * 16, 16)] = jnp.where(valid, sv & 0x3FFFF, 0)
            ef[pl.ds(g * 16, 16)] = jnp.where(valid, sv >> 18, 0)
            lx[pl.ds(g * 16, 16)] = jnp.where(valid, lg, TRASH)
        pltpu.async_copy(h_hbm.at[sx], rows_b, semg)

    def _finish(bi):
        rows_b, sx, lx, ef, semg, semc = bufs[bi]
        pltpu.make_async_copy(h_hbm.at[sx], rows_b, semg).wait()

        def _relu(r8, cc):
            es = []
            for u in range(8):
                es.append(plsc.load_gather(
                    ef, [jnp.broadcast_to(r8 * 8 + u, (16,))]))
            for u in range(8):
                r = r8 * 8 + u
                for g in range(H // 16):
                    col = g * 16 + iota16
                    a = rows_b[r, pl.ds(g * 16, 16)]
                    b = plsc.load_gather(bond_v, [es[u], col])
                    rows_b[r, pl.ds(g * 16, 16)] = jnp.maximum(a + b, 0.0)
            return cc
        lax.fori_loop(0, 16, _relu, jnp.int32(0))
        pltpu.async_copy(rows_b, acc_sh.at[lx], semc, add=True)

    def _finish_par(j):
        @pl.when(j % 2 == 0)
        def _():
            _finish(0)

        @pl.when(j % 2 == 1)
        def _():
            _finish(1)

    def _fire_step(qbase, nvalid, fcur):
        @pl.when(fcur % 2 == 0)
        def _():
            _issue(qbase, nvalid, fcur, 0)

        @pl.when(fcur % 2 == 1)
        def _():
            _issue(qbase, nvalid, fcur, 1)

        @pl.when(fcur >= 1)
        def _():
            _finish_par(fcur - 1)

    def _pass(p, carry):
        base = (NC * p + c) * R_ROWS
        _stage_issue(jnp.int32(0))

        def _zb(i, carry0):
            for g in range(H // 16):
                rows0[i, pl.ds(g * 16, 16)] = jnp.zeros((16,), jnp.float32)
            return carry0
        lax.fori_loop(0, 128, _zb, jnp.int32(0))
        nblk = R_ROWS // WBLK
        zcps = []
        for j in range(-(-nblk // NS)):
            blk = s + j * NS
            @pl.when(blk < nblk)
            def _():
                zcps.append(pltpu.async_copy(
                    rows0, acc_sh.at[pl.ds(blk * WBLK, WBLK)], sem_z))
        for j in range(-(-nblk // NS)):
            blk = s + j * NS
            @pl.when(blk < nblk)
            def _():
                pltpu.make_async_copy(
                    rows0, acc_sh.at[pl.ds(blk * WBLK, WBLK)], sem_z).wait()
        plsc.subcore_barrier()

        def _chunk(ci, carry2):
            qv_in, fc_in = carry2
            boff = (ci % 2) * EC
            _stage_wait(ci)

            @pl.when(ci + 1 < NCHUNK)
            def _():
                _stage_issue(ci + 1)

            def _scan(g5, qv):
                locs, masks, svs, cums = [], [], [], []
                for u in range(5):
                    off = boff + (g5 * 5 + u) * 16
                    d = st_dst[pl.ds(off, 16)]
                    loc = d - base
                    m = (loc >= 0) & (loc < R_ROWS)
                    locs.append(loc)
                    masks.append(m)
                    svs.append(st_se[pl.ds(off, 16)])
                    cums.append(plsc.cumsum(m.astype(jnp.int32)))
                for u in range(5):
                    pos = qv + cums[u] - 1
                    plsc.store_scatter(q_se, [pos], svs[u], mask=masks[u])
                    plsc.store_scatter(q_loc, [pos], locs[u], mask=masks[u])
                    qv = qv + plsc.all_reduce_population_count(masks[u])
                return qv
            qv_out = lax.fori_loop(0, EC // 80, _scan, qv_in)
            qn = qv_out[0]

            nf = qn // 128

            def _df(i, fc2):
                _fire_step(i * 128, 128, fc2)
                return fc2 + 1
            fc_out = lax.fori_loop(0, nf, _df, fc_in)
            rem = qn - nf * 128
            for g in range(8):
                sv = q_se[pl.ds(nf * 128 + g * 16, 16)]
                lv = q_loc[pl.ds(nf * 128 + g * 16, 16)]
                q_se[pl.ds(g * 16, 16)] = sv
                q_loc[pl.ds(g * 16, 16)] = lv
            return (jnp.broadcast_to(rem, (16,)), fc_out)
        qv_fin, fc = lax.fori_loop(0, NCHUNK, _chunk,
                                   (jnp.zeros((16,), jnp.int32), jnp.int32(0)))
        rem = qv_fin[0]

        @pl.when(rem > 0)
        def _():
            _fire_step(0, rem, fc)
        fc2 = jnp.where(rem > 0, fc + 1, fc)

        @pl.when(fc2 >= 1)
        def _():
            _finish_par(fc2 - 1)

        @pl.when(fc2 >= 1)
        def _():
            @pl.when((fc2 - 1) % 2 == 0)
            def _():
                _wait_scatter(0)

            @pl.when((fc2 - 1) % 2 == 1)
            def _():
                _wait_scatter(1)

        @pl.when(fc2 >= 2)
        def _():
            @pl.when((fc2 - 2) % 2 == 0)
            def _():
                _wait_scatter(0)

            @pl.when((fc2 - 2) % 2 == 1)
            def _():
                _wait_scatter(1)
        plsc.subcore_barrier()

        nvb = jnp.clip((SK - base) // WBLK, 0, R_ROWS // WBLK)
        for j in range(-(-(R_ROWS // WBLK) // NS)):
            blk = s + j * NS
            @pl.when(blk < nvb)
            def _():
                pltpu.async_copy(acc_sh.at[pl.ds(blk * WBLK, WBLK)],
                                 agg_hbm.at[pl.ds(base + blk * WBLK, WBLK)],
                                 sem_w)
        for j in range(-(-(R_ROWS // WBLK) // NS)):
            blk = s + j * NS
            @pl.when(blk < nvb)
            def _():
                pltpu.make_async_copy(
                    acc_sh.at[pl.ds(blk * WBLK, WBLK)],
                    agg_hbm.at[pl.ds(base + blk * WBLK, WBLK)],
                    sem_w).wait()
        plsc.subcore_barrier()
        return carry
    lax.fori_loop(0, N_PASSES, _pass, jnp.int32(0))


_aggregate = functools.partial(
    pl.kernel, _agg_body, mesh=_mesh,
    compiler_params=pltpu.CompilerParams(needs_layout_passes=False),
    out_type=jax.ShapeDtypeStruct((SK, H), jnp.float32),
    scratch_types=[
        pltpu.VMEM((2 * EC,), jnp.int32),
        pltpu.VMEM((2 * EC,), jnp.int32),
        pltpu.VMEM((QCAP,), jnp.int32),
        pltpu.VMEM((QCAP,), jnp.int32),
        pltpu.VMEM((128, H), jnp.float32),
        pltpu.VMEM((128, H), jnp.float32),
        pltpu.VMEM((EDGE_DIM, H), jnp.float32),
        pltpu.VMEM((128,), jnp.int32),
        pltpu.VMEM((128,), jnp.int32),
        pltpu.VMEM((128,), jnp.int32),
        pltpu.VMEM((128,), jnp.int32),
        pltpu.VMEM((128,), jnp.int32),
        pltpu.VMEM((128,), jnp.int32),
        pltpu.VMEM_SHARED((R_ROWS + 16, H), jnp.float32),
        pltpu.SemaphoreType.DMA,
        pltpu.SemaphoreType.DMA,
        pltpu.SemaphoreType.DMA,
        pltpu.SemaphoreType.DMA,
        pltpu.SemaphoreType.DMA,
        pltpu.SemaphoreType.DMA,
        pltpu.SemaphoreType.DMA,
        pltpu.SemaphoreType.DMA,
    ])()


MLP_BLK = 4000


def _dot(a, b):
    return lax.dot_general(a, b, (((1,), (0,)), ((), ())),
                           preferred_element_type=jnp.float32,
                           precision=lax.Precision.DEFAULT)


def _mlp_body(eps_ref, h_ref, agg_ref, w1_ref, b1_ref, w2_ref, b2_ref, out_ref):
    h = h_ref[...]
    z = (1.0 + eps_ref[0]) * h + agg_ref[...]
    z = jnp.maximum(_dot(z, w1_ref[...]) + b1_ref[...], 0.0)
    z = _dot(z, w2_ref[...]) + b2_ref[...]
    out_ref[...] = h + z


def _mlp(h, agg, w1, b1, w2, b2, eps):
    return pl.pallas_call(
        _mlp_body,
        grid=(SK // MLP_BLK,),
        in_specs=[
            pl.BlockSpec(memory_space=pltpu.SMEM),
            pl.BlockSpec((MLP_BLK, H), lambda i: (i, 0)),
            pl.BlockSpec((MLP_BLK, H), lambda i: (i, 0)),
            pl.BlockSpec((H, H), lambda i: (0, 0)),
            pl.BlockSpec((H,), lambda i: (0,)),
            pl.BlockSpec((H, H), lambda i: (0, 0)),
            pl.BlockSpec((H,), lambda i: (0,)),
        ],
        out_specs=pl.BlockSpec((MLP_BLK, H), lambda i: (i, 0)),
        out_shape=jax.ShapeDtypeStruct((SK, H), jnp.float32),
    )(eps.reshape(1), h, agg, w1, b1, w2, b2)


EPI_NODES = 250
EPI_ROWS = EPI_NODES * M * K
EPI_GRID = N_TOTAL // EPI_NODES


def _epi_body(alpha_ref, h_ref, lp_ref, b_ref, out_ref):
    i = pl.program_id(0)
    lp = lp_ref[0]
    lp = jnp.where(jnp.isfinite(lp), lp, 0.0)
    t = -alpha_ref[0] * lp
    t = t - jnp.max(t, axis=1, keepdims=True)
    e = jnp.exp(t)
    wgt = e / jnp.sum(e, axis=1, keepdims=True)
    n_idx = lax.broadcasted_iota(jnp.int32, (EPI_NODES, EPI_ROWS), 0)
    r_idx = lax.broadcasted_iota(jnp.int32, (EPI_NODES, EPI_ROWS), 1)
    match = (r_idx // (M * K)) == n_idx
    slot1 = ((r_idx // K) % M) == 1
    val = jnp.where(slot1, wgt[:, 1:2], wgt[:, 0:1]) * (1.0 / K)
    sel = jnp.where(match, val, 0.0)
    ne = _dot(sel, h_ref[...])
    bcol = b_ref[0, 0, :]
    oh = (bcol[:, None] ==
          lax.broadcasted_iota(jnp.int32, (EPI_NODES, B_GRAPHS), 1)
          ).astype(jnp.float32)
    contrib = lax.dot_general(oh, ne, (((0,), (0,)), ((), ())),
                              preferred_element_type=jnp.float32,
                              precision=lax.Precision.DEFAULT)

    @pl.when(i == 0)
    def _():
        out_ref[...] = contrib

    @pl.when(i > 0)
    def _():
        out_ref[...] += contrib


def _epilogue(h, lp2, batch2, alpha):
    return pl.pallas_call(
        _epi_body,
        grid=(EPI_GRID,),
        in_specs=[
            pl.BlockSpec(memory_space=pltpu.SMEM),
            pl.BlockSpec((EPI_ROWS, H), lambda i: (i, 0)),
            pl.BlockSpec((1, EPI_NODES, M), lambda i: (i, 0, 0)),
            pl.BlockSpec((1, 1, EPI_NODES), lambda i: (i, 0, 0)),
        ],
        out_specs=pl.BlockSpec((B_GRAPHS, H), lambda i: (0, 0)),
        out_shape=jax.ShapeDtypeStruct((B_GRAPHS, H), jnp.float32),
    )(alpha, h, lp2, batch2)


def kernel(x, nodes_sampled, log_probs, intra_ei, edge_attr, batch,
           atom_table, bond_table, role_table, W1, b1, W2, b2, eps, ht_alpha):
    x32 = x.astype(jnp.int32)
    nid = nodes_sampled.reshape(-1).astype(jnp.int32)
    src = intra_ei[0].astype(jnp.int32)
    dst = intra_ei[1].astype(jnp.int32)
    ea0 = edge_attr.astype(jnp.int32) - 1
    se_pk = src | (ea0 << 18)
    comb = (role_table[:, None, :] + atom_table[None, :, :]).reshape(
        2 * IN_CH, H)
    comb = jnp.tile(comb, (8, 1))
    h = _prologue(comb, x32, nid)
    for l in range(W1.shape[0]):
        agg = _aggregate(h, se_pk, dst, bond_table)
        h = _mlp(h, agg, W1[l], b1[l], W2[l], b2[l], eps[l])
    lp3 = log_probs.reshape(EPI_GRID, EPI_NODES, M)
    batch3 = batch.astype(jnp.int32).reshape(EPI_GRID, 1, EPI_NODES)
    return _epilogue(h, lp3, batch3, ht_alpha)


# speedup vs baseline: 1.0098x; 1.0098x over previous
import functools

import jax
import jax.numpy as jnp
from jax import lax
from jax.experimental import pallas as pl
from jax.experimental.pallas import tpu as pltpu
from jax.experimental.pallas import tpu_sc as plsc

N_TOTAL = 10000
M = 2
S = N_TOTAL * M
K = 8
SK = S * K
E = 320000
H = 128
IN_CH = 128
EDGE_DIM = 16
B_GRAPHS = 64

NC = 2
NS = 16
NW = NC * NS

R_ROWS = 10240
N_PASSES = -(-SK // (NC * R_ROWS))
TRASH = R_ROWS
E_TILE = E // NS
EC = 2000
NCHUNK = E_TILE // EC
QCAP = EC + 160
WBLK = 128

_mesh = plsc.VectorSubcoreMesh(core_axis_name="c", subcore_axis_name="s",
                               num_cores=NC, num_subcores=NS)


PRO_CHUNK = 200
PRO_PER_W = SK // NW
PRO_NCHUNK = PRO_PER_W // PRO_CHUNK


def _pro_body(comb_hbm, x_hbm, nid_hbm, h0_hbm, x_res, nid_st,
              idx0, idx1, rows0, rows1,
              semn, semg0, semg1, semw0, semw1):
    c = lax.axis_index("c")
    s = lax.axis_index("s")
    w = s * NC + c
    base = w * PRO_PER_W
    pltpu.sync_copy(x_hbm, x_res)
    iota16 = lax.iota(jnp.int32, 16)
    rootpat = (iota16 % K) == 0
    bufs = ((idx0, rows0, semg0, semw0), (idx1, rows1, semg1, semw1))

    def _nid_refs(ci):
        return (nid_hbm.at[pl.ds(base + ci * PRO_CHUNK, PRO_CHUNK)],
                nid_st.at[pl.ds((ci % 2) * 208, PRO_CHUNK)], semn)

    def _gather_cps(ci):
        idxb, rowsb, semg, _ = bufs[ci % 2]
        return ((comb_hbm.at[idxb.at[pl.ds(0, 128)]],
                 rowsb.at[pl.ds(0, 128)], semg),
                (comb_hbm.at[idxb.at[pl.ds(128, PRO_CHUNK - 128)]],
                 rowsb.at[pl.ds(128, PRO_CHUNK - 128)], semg))

    def _wr_refs(ci):
        rowsb, semw = bufs[ci % 2][1], bufs[ci % 2][3]
        return (rowsb, h0_hbm.at[pl.ds(base + ci * PRO_CHUNK, PRO_CHUNK)], semw)

    sre, dst0, sm = _nid_refs(0)
    pltpu.async_copy(sre, dst0, sm)
    for ci in range(PRO_NCHUNK):
        b = ci % 2
        idxb = bufs[b][0]
        sre, dstn, sm = _nid_refs(ci)
        pltpu.make_async_copy(sre, dstn, sm).wait()
        if ci + 1 < PRO_NCHUNK:
            sre, dstn, sm = _nid_refs(ci + 1)
            pltpu.async_copy(sre, dstn, sm)
        rep = (w % 8) * (2 * IN_CH)
        for g in range(13):
            nv = nid_st[pl.ds(b * 208 + g * 16, 16)]
            if g == 12:
                nv = jnp.where(iota16 < 8, nv, 0)
            xv = plsc.load_gather(x_res, [nv])
            idxb[pl.ds(g * 16, 16)] = xv + jnp.where(rootpat, IN_CH, 0) + rep
        if ci >= 2:
            sre, dstn, sm = _wr_refs(ci - 2)
            pltpu.make_async_copy(sre, dstn, sm).wait()
        for sre, dstn, sm in _gather_cps(ci):
            pltpu.async_copy(sre, dstn, sm)
        if ci >= 1:
            for sre, dstn, sm in _gather_cps(ci - 1):
                pltpu.make_async_copy(sre, dstn, sm).wait()
            sre, dstn, sm = _wr_refs(ci - 1)
            pltpu.async_copy(sre, dstn, sm)
    last = PRO_NCHUNK - 1
    for sre, dstn, sm in _gather_cps(last):
        pltpu.make_async_copy(sre, dstn, sm).wait()
    sre, dstn, sm = _wr_refs(last)
    pltpu.async_copy(sre, dstn, sm)
    for ci in (last - 1, last):
        sre, dstn, sm = _wr_refs(ci)
        pltpu.make_async_copy(sre, dstn, sm).wait()


_prologue = functools.partial(
    pl.kernel, _pro_body, mesh=_mesh,
    compiler_params=pltpu.CompilerParams(needs_layout_passes=False),
    out_type=jax.ShapeDtypeStruct((SK, H), jnp.float32),
    scratch_types=[
        pltpu.VMEM((N_TOTAL,), jnp.int32),
        pltpu.VMEM((416,), jnp.int32),
        pltpu.VMEM((PRO_CHUNK + 8,), jnp.int32),
        pltpu.VMEM((PRO_CHUNK + 8,), jnp.int32),
        pltpu.VMEM((PRO_CHUNK, H), jnp.float32),
        pltpu.VMEM((PRO_CHUNK, H), jnp.float32),
        pltpu.SemaphoreType.DMA,
        pltpu.SemaphoreType.DMA,
        pltpu.SemaphoreType.DMA,
        pltpu.SemaphoreType.DMA,
        pltpu.SemaphoreType.DMA,
    ])()


def _agg_body(h_hbm, se_hbm, dst_hbm, bond_hbm, agg_hbm,
              st_se, st_dst, q_se, q_loc, rows0, rows1, bond_v,
              sx0, sx1, lx0, lx1, ef0, ef1, acc_sh,
              semg0, semg1, semc0, semc1, sem_s1, sem_s2, sem_z, sem_w):
    c = lax.axis_index("c")
    s = lax.axis_index("s")
    ebase = s * E_TILE
    iota16 = lax.iota(jnp.int32, 16)
    pltpu.sync_copy(bond_hbm, bond_v)
    bufs = ((rows0, sx0, lx0, ef0, semg0, semc0),
            (rows1, sx1, lx1, ef1, semg1, semc1))

    def _stage_refs(ci):
        boff = (ci % 2) * EC
        sl = pl.ds(ebase + ci * EC, EC)
        bl = pl.ds(boff, EC)
        return ((se_hbm.at[sl], st_se.at[bl], sem_s1),
                (dst_hbm.at[sl], st_dst.at[bl], sem_s2))

    def _stage_issue(ci):
        for src, dst, sem in _stage_refs(ci):
            pltpu.async_copy(src, dst, sem)

    def _stage_wait(ci):
        for src, dst, sem in _stage_refs(ci):
            pltpu.make_async_copy(src, dst, sem).wait()

    def _wait_scatter(bi):
        rows_b, _, lx, _, _, semc = bufs[bi]
        pltpu.make_async_copy(rows_b, acc_sh.at[lx], semc).wait()

    def _issue(qbase, nvalid, j, bi):
        rows_b, sx, lx, ef, semg, _ = bufs[bi]

        @pl.when(j >= 2)
        def _():
            _wait_scatter(bi)
        for g in range(8):
            lane = g * 16 + iota16
            valid = lane < nvalid
            sv = q_se[pl.ds(qbase + g * 16, 16)]
            lg = q_loc[pl.ds(qbase + g * 16, 16)]
            sx[pl.ds(g * 16, 16)] = jnp.where(valid, sv & 0x3FFFF, 0)
            ef[pl.ds(g * 16, 16)] = jnp.where(valid, sv >> 18, 0)
            lx[pl.ds(g * 16, 16)] = jnp.where(valid, lg, TRASH)
        pltpu.async_copy(h_hbm.at[sx], rows_b, semg)

    def _finish(bi):
        rows_b, sx, lx, ef, semg, semc = bufs[bi]
        pltpu.make_async_copy(h_hbm.at[sx], rows_b, semg).wait()

        def _relu(r4, cc):
            es = []
            for u in range(4):
                es.append(plsc.load_gather(
                    ef, [jnp.broadcast_to(r4 * 4 + u, (16,))]))
            for u in range(4):
                r = r4 * 4 + u
                for g in range(H // 16):
                    col = g * 16 + iota16
                    a = rows_b[r, pl.ds(g * 16, 16)]
                    b = plsc.load_gather(bond_v, [es[u], col])
                    rows_b[r, pl.ds(g * 16, 16)] = jnp.maximum(a + b, 0.0)
            return cc
        lax.fori_loop(0, 32, _relu, jnp.int32(0))
        pltpu.async_copy(rows_b, acc_sh.at[lx], semc, add=True)

    def _finish_par(j):
        @pl.when(j % 2 == 0)
        def _():
            _finish(0)

        @pl.when(j % 2 == 1)
        def _():
            _finish(1)

    def _fire_step(qbase, nvalid, fcur):
        @pl.when(fcur % 2 == 0)
        def _():
            _issue(qbase, nvalid, fcur, 0)

        @pl.when(fcur % 2 == 1)
        def _():
            _issue(qbase, nvalid, fcur, 1)

        @pl.when(fcur >= 1)
        def _():
            _finish_par(fcur - 1)

    def _pass(p, carry):
        base = (NC * p + c) * R_ROWS
        _stage_issue(jnp.int32(0))

        def _zb(i, carry0):
            for g in range(H // 16):
                rows0[i, pl.ds(g * 16, 16)] = jnp.zeros((16,), jnp.float32)
            return carry0
        lax.fori_loop(0, 128, _zb, jnp.int32(0))
        nblk = R_ROWS // WBLK
        zcps = []
        for j in range(-(-nblk // NS)):
            blk = s + j * NS
            @pl.when(blk < nblk)
            def _():
                zcps.append(pltpu.async_copy(
                    rows0, acc_sh.at[pl.ds(blk * WBLK, WBLK)], sem_z))
        for j in range(-(-nblk // NS)):
            blk = s + j * NS
            @pl.when(blk < nblk)
            def _():
                pltpu.make_async_copy(
                    rows0, acc_sh.at[pl.ds(blk * WBLK, WBLK)], sem_z).wait()
        plsc.subcore_barrier()

        def _chunk(ci, carry2):
            qv_in, fc_in = carry2
            boff = (ci % 2) * EC
            _stage_wait(ci)

            @pl.when(ci + 1 < NCHUNK)
            def _():
                _stage_issue(ci + 1)

            def _scan(g5, qv):
                locs, masks, svs, cums = [], [], [], []
                for u in range(5):
                    off = boff + (g5 * 5 + u) * 16
                    d = st_dst[pl.ds(off, 16)]
                    loc = d - base
                    m = (loc >= 0) & (loc < R_ROWS)
                    locs.append(loc)
                    masks.append(m)
                    svs.append(st_se[pl.ds(off, 16)])
                    cums.append(plsc.cumsum(m.astype(jnp.int32)))
                for u in range(5):
                    pos = qv + cums[u] - 1
                    plsc.store_scatter(q_se, [pos], svs[u], mask=masks[u])
                    plsc.store_scatter(q_loc, [pos], locs[u], mask=masks[u])
                    qv = qv + plsc.all_reduce_population_count(masks[u])
                return qv
            qv_out = lax.fori_loop(0, EC // 80, _scan, qv_in)
            qn = qv_out[0]

            nf = qn // 128

            def _df(i, fc2):
                _fire_step(i * 128, 128, fc2)
                return fc2 + 1
            fc_out = lax.fori_loop(0, nf, _df, fc_in)
            rem = qn - nf * 128
            for g in range(8):
                sv = q_se[pl.ds(nf * 128 + g * 16, 16)]
                lv = q_loc[pl.ds(nf * 128 + g * 16, 16)]
                q_se[pl.ds(g * 16, 16)] = sv
                q_loc[pl.ds(g * 16, 16)] = lv
            return (jnp.broadcast_to(rem, (16,)), fc_out)
        qv_fin, fc = lax.fori_loop(0, NCHUNK, _chunk,
                                   (jnp.zeros((16,), jnp.int32), jnp.int32(0)))
        rem = qv_fin[0]

        @pl.when(rem > 0)
        def _():
            _fire_step(0, rem, fc)
        fc2 = jnp.where(rem > 0, fc + 1, fc)

        @pl.when(fc2 >= 1)
        def _():
            _finish_par(fc2 - 1)

        @pl.when(fc2 >= 1)
        def _():
            @pl.when((fc2 - 1) % 2 == 0)
            def _():
                _wait_scatter(0)

            @pl.when((fc2 - 1) % 2 == 1)
            def _():
                _wait_scatter(1)

        @pl.when(fc2 >= 2)
        def _():
            @pl.when((fc2 - 2) % 2 == 0)
            def _():
                _wait_scatter(0)

            @pl.when((fc2 - 2) % 2 == 1)
            def _():
                _wait_scatter(1)
        plsc.subcore_barrier()

        nvb = jnp.clip((SK - base) // WBLK, 0, R_ROWS // WBLK)
        for j in range(-(-(R_ROWS // WBLK) // NS)):
            blk = s + j * NS
            @pl.when(blk < nvb)
            def _():
                pltpu.async_copy(acc_sh.at[pl.ds(blk * WBLK, WBLK)],
                                 agg_hbm.at[pl.ds(base + blk * WBLK, WBLK)],
                                 sem_w)
        for j in range(-(-(R_ROWS // WBLK) // NS)):
            blk = s + j * NS
            @pl.when(blk < nvb)
            def _():
                pltpu.make_async_copy(
                    acc_sh.at[pl.ds(blk * WBLK, WBLK)],
                    agg_hbm.at[pl.ds(base + blk * WBLK, WBLK)],
                    sem_w).wait()
        plsc.subcore_barrier()
        return carry
    lax.fori_loop(0, N_PASSES, _pass, jnp.int32(0))


_aggregate = functools.partial(
    pl.kernel, _agg_body, mesh=_mesh,
    compiler_params=pltpu.CompilerParams(needs_layout_passes=False),
    out_type=jax.ShapeDtypeStruct((SK, H), jnp.float32),
    scratch_types=[
        pltpu.VMEM((2 * EC,), jnp.int32),
        pltpu.VMEM((2 * EC,), jnp.int32),
        pltpu.VMEM((QCAP,), jnp.int32),
        pltpu.VMEM((QCAP,), jnp.int32),
        pltpu.VMEM((128, H), jnp.float32),
        pltpu.VMEM((128, H), jnp.float32),
        pltpu.VMEM((EDGE_DIM, H), jnp.float32),
        pltpu.VMEM((128,), jnp.int32),
        pltpu.VMEM((128,), jnp.int32),
        pltpu.VMEM((128,), jnp.int32),
        pltpu.VMEM((128,), jnp.int32),
        pltpu.VMEM((128,), jnp.int32),
        pltpu.VMEM((128,), jnp.int32),
        pltpu.VMEM_SHARED((R_ROWS + 16, H), jnp.float32),
        pltpu.SemaphoreType.DMA,
        pltpu.SemaphoreType.DMA,
        pltpu.SemaphoreType.DMA,
        pltpu.SemaphoreType.DMA,
        pltpu.SemaphoreType.DMA,
        pltpu.SemaphoreType.DMA,
        pltpu.SemaphoreType.DMA,
        pltpu.SemaphoreType.DMA,
    ])()


MLP_BLK = 4000


def _dot(a, b):
    return lax.dot_general(a, b, (((1,), (0,)), ((), ())),
                           preferred_element_type=jnp.float32,
                           precision=lax.Precision.DEFAULT)


def _mlp_body(eps_ref, h_ref, agg_ref, w1_ref, b1_ref, w2_ref, b2_ref, out_ref):
    h = h_ref[...]
    z = (1.0 + eps_ref[0]) * h + agg_ref[...]
    z = jnp.maximum(_dot(z, w1_ref[...]) + b1_ref[...], 0.0)
    z = _dot(z, w2_ref[...]) + b2_ref[...]
    out_ref[...] = h + z


def _mlp(h, agg, w1, b1, w2, b2, eps):
    return pl.pallas_call(
        _mlp_body,
        grid=(SK // MLP_BLK,),
        in_specs=[
            pl.BlockSpec(memory_space=pltpu.SMEM),
            pl.BlockSpec((MLP_BLK, H), lambda i: (i, 0)),
            pl.BlockSpec((MLP_BLK, H), lambda i: (i, 0)),
            pl.BlockSpec((H, H), lambda i: (0, 0)),
            pl.BlockSpec((H,), lambda i: (0,)),
            pl.BlockSpec((H, H), lambda i: (0, 0)),
            pl.BlockSpec((H,), lambda i: (0,)),
        ],
        out_specs=pl.BlockSpec((MLP_BLK, H), lambda i: (i, 0)),
        out_shape=jax.ShapeDtypeStruct((SK, H), jnp.float32),
    )(eps.reshape(1), h, agg, w1, b1, w2, b2)


EPI_NODES = 250
EPI_ROWS = EPI_NODES * M * K
EPI_GRID = N_TOTAL // EPI_NODES


def _epi_body(alpha_ref, h_ref, lp_ref, b_ref, out_ref):
    i = pl.program_id(0)
    lp = lp_ref[0]
    lp = jnp.where(jnp.isfinite(lp), lp, 0.0)
    t = -alpha_ref[0] * lp
    t = t - jnp.max(t, axis=1, keepdims=True)
    e = jnp.exp(t)
    wgt = e / jnp.sum(e, axis=1, keepdims=True)
    n_idx = lax.broadcasted_iota(jnp.int32, (EPI_NODES, EPI_ROWS), 0)
    r_idx = lax.broadcasted_iota(jnp.int32, (EPI_NODES, EPI_ROWS), 1)
    match = (r_idx // (M * K)) == n_idx
    slot1 = ((r_idx // K) % M) == 1
    val = jnp.where(slot1, wgt[:, 1:2], wgt[:, 0:1]) * (1.0 / K)
    sel = jnp.where(match, val, 0.0)
    ne = _dot(sel, h_ref[...])
    bcol = b_ref[0, 0, :]
    oh = (bcol[:, None] ==
          lax.broadcasted_iota(jnp.int32, (EPI_NODES, B_GRAPHS), 1)
          ).astype(jnp.float32)
    contrib = lax.dot_general(oh, ne, (((0,), (0,)), ((), ())),
                              preferred_element_type=jnp.float32,
                              precision=lax.Precision.DEFAULT)

    @pl.when(i == 0)
    def _():
        out_ref[...] = contrib

    @pl.when(i > 0)
    def _():
        out_ref[...] += contrib


def _epilogue(h, lp2, batch2, alpha):
    return pl.pallas_call(
        _epi_body,
        grid=(EPI_GRID,),
        in_specs=[
            pl.BlockSpec(memory_space=pltpu.SMEM),
            pl.BlockSpec((EPI_ROWS, H), lambda i: (i, 0)),
            pl.BlockSpec((1, EPI_NODES, M), lambda i: (i, 0, 0)),
            pl.BlockSpec((1, 1, EPI_NODES), lambda i: (i, 0, 0)),
        ],
        out_specs=pl.BlockSpec((B_GRAPHS, H), lambda i: (0, 0)),
        out_shape=jax.ShapeDtypeStruct((B_GRAPHS, H), jnp.float32),
    )(alpha, h, lp2, batch2)


def kernel(x, nodes_sampled, log_probs, intra_ei, edge_attr, batch,
           atom_table, bond_table, role_table, W1, b1, W2, b2, eps, ht_alpha):
    x32 = x.astype(jnp.int32)
    nid = nodes_sampled.reshape(-1).astype(jnp.int32)
    src = intra_ei[0].astype(jnp.int32)
    dst = intra_ei[1].astype(jnp.int32)
    ea0 = edge_attr.astype(jnp.int32) - 1
    se_pk = src | (ea0 << 18)
    comb = (role_table[:, None, :] + atom_table[None, :, :]).reshape(
        2 * IN_CH, H)
    comb = jnp.tile(comb, (8, 1))
    h = _prologue(comb, x32, nid)
    for l in range(W1.shape[0]):
        agg = _aggregate(h, se_pk, dst, bond_table)
        h = _mlp(h, agg, W1[l], b1[l], W2[l], b2[l], eps[l])
    lp3 = log_probs.reshape(EPI_GRID, EPI_NODES, M)
    batch3 = batch.astype(jnp.int32).reshape(EPI_GRID, 1, EPI_NODES)
    return _epilogue(h, lp3, batch3, ht_alpha)
